# Initial kernel scaffold; baseline (speedup 1.0000x reference)
import functools

import jax
import jax.numpy as jnp
from jax import lax
from jax.experimental import pallas as pl
from jax.experimental.pallas import tpu as pltpu
from jax.experimental.pallas import tpu_sc as plsc

L = 16
H = 128
HC = H // L
S = 200
B = 4096
NC = 2
NS = 16
NW = NC * NS
BPW = B // NW
SHALF = S // 2


def _rsqrt_newton(x):
    i = lax.bitcast_convert_type(x, jnp.int32)
    i = jnp.int32(0x5F3759DF) - lax.shift_right_logical(i, 1)
    y = lax.bitcast_convert_type(i, jnp.float32)
    xh = x * jnp.float32(0.5)
    for _ in range(3):
        y = y * (jnp.float32(1.5) - xh * y * y)
    return y


def _make_kernel():
    mesh = plsc.VectorSubcoreMesh(core_axis_name="c", subcore_axis_name="s")

    @functools.partial(
        pl.kernel,
        out_type=jax.ShapeDtypeStruct((B, S, H), jnp.float32),
        mesh=mesh,
        scratch_types=[
            pltpu.VMEM((2, SHALF), jnp.int32),
            pltpu.VMEM((S, H), jnp.float32),
            pltpu.VMEM((S, H), jnp.float32),
            pltpu.VMEM((H,), jnp.float32),
            pltpu.VMEM((H,), jnp.float32),
            pltpu.SemaphoreType.DMA,
        ],
    )
    def k(ids_hbm, word_hbm, pos_hbm, w_hbm, b_hbm, out_hbm,
          idx_v, buf, pos_v, w_v, b_v, sem):
        wid = lax.axis_index("s") * NC + lax.axis_index("c")

        pltpu.sync_copy(pos_hbm.at[pl.ds(0, S)], pos_v)
        pltpu.sync_copy(w_hbm, w_v)
        pltpu.sync_copy(b_hbm, b_v)

        w_regs = [w_v[pl.ds(j * L, L)] for j in range(HC)]
        b_regs = [b_v[pl.ds(j * L, L)] for j in range(HC)]

        def batch_body(t, carry):
            bi = wid * BPW + t
            pltpu.sync_copy(ids_hbm.at[bi], idx_v)
            cp0 = pltpu.async_copy(
                word_hbm.at[idx_v.at[0]], buf.at[pl.ds(0, SHALF)], sem)
            cp1 = pltpu.async_copy(
                word_hbm.at[idx_v.at[1]], buf.at[pl.ds(SHALF, SHALF)], sem)
            cp0.wait()
            cp1.wait()

            def row_body(i, c):
                x = [buf[i, pl.ds(j * L, L)] + pos_v[i, pl.ds(j * L, L)]
                     for j in range(HC)]
                s = (x[0] + x[1]) + (x[2] + x[3])
                s = s + (x[4] + x[5]) + (x[6] + x[7])
                q = x[0] * x[0] + x[1] * x[1]
                q = q + x[2] * x[2] + x[3] * x[3]
                q = q + x[4] * x[4] + x[5] * x[5]
                q = q + x[6] * x[6] + x[7] * x[7]
                tot = jnp.sum(s)
                tot2 = jnp.sum(q)
                mean = tot * jnp.float32(1.0 / H)
                var = tot2 * jnp.float32(1.0 / H) - mean * mean
                inv = _rsqrt_newton(var + jnp.float32(1e-6))
                for j in range(HC):
                    y = (x[j] - mean) * inv * w_regs[j] + b_regs[j]
                    buf[i, pl.ds(j * L, L)] = y
                return c

            lax.fori_loop(0, S, row_body, 0, unroll=False)
            pltpu.sync_copy(buf, out_hbm.at[bi])
            return carry

        lax.fori_loop(0, BPW, batch_body, 0, unroll=False)

    return k


_kernel_call = _make_kernel()


def kernel(input_ids, word_emb, pos_emb, ln_weight, ln_bias):
    ids3 = input_ids.astype(jnp.int32).reshape(B, 2, SHALF)
    return _kernel_call(ids3, word_emb, pos_emb, ln_weight, ln_bias)


# speedup vs baseline: 2.0766x; 2.0766x over previous
import functools

import jax
import jax.numpy as jnp
from jax import lax
from jax.experimental import pallas as pl
from jax.experimental.pallas import tpu as pltpu
from jax.experimental.pallas import tpu_sc as plsc

L = 16
H = 128
HC = H // L
S = 200
B = 4096
NC = 2
NS = 16
NW = NC * NS
BPW = B // NW
SHALF = S // 2


def _rsqrt_newton(x):
    i = lax.bitcast_convert_type(x, jnp.int32)
    i = jnp.int32(0x5F3759DF) - lax.shift_right_logical(i, 1)
    y = lax.bitcast_convert_type(i, jnp.float32)
    xh = x * jnp.float32(0.5)
    for _ in range(3):
        y = y * (jnp.float32(1.5) - xh * y * y)
    return y


def _make_kernel():
    mesh = plsc.VectorSubcoreMesh(core_axis_name="c", subcore_axis_name="s")

    @functools.partial(
        pl.kernel,
        out_type=jax.ShapeDtypeStruct((B, S, H), jnp.float32),
        mesh=mesh,
        compiler_params=pltpu.CompilerParams(needs_layout_passes=False),
        scratch_types=[
            pltpu.VMEM((2, SHALF), jnp.int32),
            pltpu.VMEM((S, H), jnp.float32),
            pltpu.VMEM((S, H), jnp.float32),
            pltpu.VMEM((H,), jnp.float32),
            pltpu.VMEM((H,), jnp.float32),
            pltpu.SemaphoreType.DMA,
        ],
    )
    def k(ids_hbm, word_hbm, pos_hbm, w_hbm, b_hbm, out_hbm,
          idx_v, buf, pos_v, w_v, b_v, sem):
        wid = lax.axis_index("s") * NC + lax.axis_index("c")

        pltpu.sync_copy(pos_hbm.at[pl.ds(0, S)], pos_v)
        pltpu.sync_copy(w_hbm, w_v)
        pltpu.sync_copy(b_hbm, b_v)

        w_regs = [w_v[pl.ds(j * L, L)] for j in range(HC)]
        b_regs = [b_v[pl.ds(j * L, L)] for j in range(HC)]

        def batch_body(t, carry):
            bi = wid * BPW + t
            pltpu.sync_copy(ids_hbm.at[bi], idx_v)
            cp0 = pltpu.async_copy(
                word_hbm.at[idx_v.at[0]], buf.at[pl.ds(0, SHALF)], sem)
            cp1 = pltpu.async_copy(
                word_hbm.at[idx_v.at[1]], buf.at[pl.ds(SHALF, SHALF)], sem)
            cp0.wait()
            cp1.wait()

            def row_body(i, c):
                x = [buf[i, pl.ds(j * L, L)] + pos_v[i, pl.ds(j * L, L)]
                     for j in range(HC)]
                s = (x[0] + x[1]) + (x[2] + x[3])
                s = s + (x[4] + x[5]) + (x[6] + x[7])
                q = x[0] * x[0] + x[1] * x[1]
                q = q + x[2] * x[2] + x[3] * x[3]
                q = q + x[4] * x[4] + x[5] * x[5]
                q = q + x[6] * x[6] + x[7] * x[7]
                tot = jnp.sum(s)
                tot2 = jnp.sum(q)
                mean = tot * jnp.float32(1.0 / H)
                var = tot2 * jnp.float32(1.0 / H) - mean * mean
                inv = _rsqrt_newton(var + jnp.float32(1e-6))
                for j in range(HC):
                    y = (x[j] - mean) * inv * w_regs[j] + b_regs[j]
                    buf[i, pl.ds(j * L, L)] = y
                return c

            lax.fori_loop(0, S, row_body, 0, unroll=False)
            pltpu.sync_copy(buf, out_hbm.at[bi])
            return carry

        lax.fori_loop(0, BPW, batch_body, 0, unroll=False)

    return k


_kernel_call = _make_kernel()


def kernel(input_ids, word_emb, pos_emb, ln_weight, ln_bias):
    ids3 = input_ids.astype(jnp.int32).reshape(B, 2, SHALF)
    return _kernel_call(ids3, word_emb, pos_emb, ln_weight, ln_bias)


# speedup vs baseline: 2.5063x; 1.2069x over previous
import functools

import jax
import jax.numpy as jnp
from jax import lax
from jax.experimental import pallas as pl
from jax.experimental.pallas import tpu as pltpu
from jax.experimental.pallas import tpu_sc as plsc

L = 16
H = 128
HC = H // L
S = 200
B = 4096
NC = 2
NS = 16
NW = NC * NS
BPW = B // NW
SHALF = S // 2


def _rsqrt_newton(x):
    i = lax.bitcast_convert_type(x, jnp.int32)
    i = jnp.int32(0x5F3759DF) - lax.shift_right_logical(i, 1)
    y = lax.bitcast_convert_type(i, jnp.float32)
    xh = x * jnp.float32(0.5)
    for _ in range(3):
        y = y * (jnp.float32(1.5) - xh * y * y)
    return y


def _make_kernel():
    mesh = plsc.VectorSubcoreMesh(core_axis_name="c", subcore_axis_name="s")

    @functools.partial(
        pl.kernel,
        out_type=jax.ShapeDtypeStruct((B, S, H), jnp.float32),
        mesh=mesh,
        compiler_params=pltpu.CompilerParams(needs_layout_passes=False),
        scratch_types=[
            pltpu.VMEM((BPW, 2, SHALF), jnp.int32),
            pltpu.VMEM((S, H), jnp.float32),
            pltpu.VMEM((S, H), jnp.float32),
            pltpu.VMEM((S, H), jnp.float32),
            pltpu.VMEM((H,), jnp.float32),
            pltpu.VMEM((H,), jnp.float32),
            pltpu.SemaphoreType.DMA,
            pltpu.SemaphoreType.DMA,
        ],
    )
    def k(ids_hbm, word_hbm, pos_hbm, w_hbm, b_hbm, out_hbm,
          ids_v, buf0, buf1, pos_v, w_v, b_v, gsem, wsem):
        wid = lax.axis_index("s") * NC + lax.axis_index("c")
        base = wid * BPW

        pltpu.sync_copy(ids_hbm.at[pl.ds(base, BPW)], ids_v)
        pltpu.sync_copy(pos_hbm.at[pl.ds(0, S)], pos_v)
        pltpu.sync_copy(w_hbm, w_v)
        pltpu.sync_copy(b_hbm, b_v)

        w_regs = [w_v[pl.ds(j * L, L)] for j in range(HC)]
        b_regs = [b_v[pl.ds(j * L, L)] for j in range(HC)]

        def start_gather(t, buf):
            pltpu.async_copy(
                word_hbm.at[ids_v.at[t, 0]], buf.at[pl.ds(0, SHALF)], gsem)
            pltpu.async_copy(
                word_hbm.at[ids_v.at[t, 1]], buf.at[pl.ds(SHALF, SHALF)], gsem)

        def wait_gather(t, buf):
            pltpu.make_async_copy(
                word_hbm.at[ids_v.at[t, 0]], buf.at[pl.ds(0, SHALF)],
                gsem).wait()
            pltpu.make_async_copy(
                word_hbm.at[ids_v.at[t, 1]], buf.at[pl.ds(SHALF, SHALF)],
                gsem).wait()

        def compute(t, buf):
            def row_body(i, c):
                x = [buf[i, pl.ds(j * L, L)] + pos_v[i, pl.ds(j * L, L)]
                     for j in range(HC)]
                s = (x[0] + x[1]) + (x[2] + x[3])
                s = s + (x[4] + x[5]) + (x[6] + x[7])
                q = x[0] * x[0] + x[1] * x[1]
                q = q + x[2] * x[2] + x[3] * x[3]
                q = q + x[4] * x[4] + x[5] * x[5]
                q = q + x[6] * x[6] + x[7] * x[7]
                tot = jnp.sum(s)
                tot2 = jnp.sum(q)
                mean = tot * jnp.float32(1.0 / H)
                var = tot2 * jnp.float32(1.0 / H) - mean * mean
                inv = _rsqrt_newton(var + jnp.float32(1e-6))
                for j in range(HC):
                    y = (x[j] - mean) * inv * w_regs[j] + b_regs[j]
                    buf[i, pl.ds(j * L, L)] = y
                return c

            lax.fori_loop(0, S, row_body, 0, unroll=False)
            pltpu.async_copy(buf, out_hbm.at[base + t], wsem)

        def wait_writeback(t, buf):
            pltpu.make_async_copy(buf, out_hbm.at[base + t], wsem).wait()

        start_gather(0, buf0)
        start_gather(1, buf1)
        wait_gather(0, buf0)
        compute(0, buf0)

        def pair(kk, c):
            t1 = 2 * kk + 1
            wait_writeback(t1 - 1, buf0)
            start_gather(t1 + 1, buf0)
            wait_gather(t1, buf1)
            compute(t1, buf1)
            t2 = t1 + 1
            wait_writeback(t2 - 1, buf1)
            start_gather(t2 + 1, buf1)
            wait_gather(t2, buf0)
            compute(t2, buf0)
            return c

        lax.fori_loop(0, (BPW - 2) // 2, pair, 0, unroll=False)

        t_last = BPW - 1
        wait_gather(t_last, buf1)
        compute(t_last, buf1)
        wait_writeback(t_last - 1, buf0)
        wait_writeback(t_last, buf1)

    return k


_kernel_call = _make_kernel()


def kernel(input_ids, word_emb, pos_emb, ln_weight, ln_bias):
    ids3 = input_ids.astype(jnp.int32).reshape(B, 2, SHALF)
    return _kernel_call(ids3, word_emb, pos_emb, ln_weight, ln_bias)


# speedup vs baseline: 5.7584x; 2.2976x over previous
import functools

import jax
import jax.numpy as jnp
from jax import lax
from jax.experimental import pallas as pl
from jax.experimental.pallas import tpu as pltpu
from jax.experimental.pallas import tpu_sc as plsc

L = 16
H = 128
HC = H // L
S = 200
B = 4096
NC = 2
NS = 16
NW = NC * NS
BPW = B // NW
SHALF = S // 2


def _rsqrt_newton(x):
    i = lax.bitcast_convert_type(x, jnp.int32)
    i = jnp.int32(0x5F3759DF) - lax.shift_right_logical(i, 1)
    y = lax.bitcast_convert_type(i, jnp.float32)
    xh = x * jnp.float32(0.5)
    for _ in range(3):
        y = y * (jnp.float32(1.5) - xh * y * y)
    return y


def _make_kernel():
    mesh = plsc.VectorSubcoreMesh(core_axis_name="c", subcore_axis_name="s")

    @functools.partial(
        pl.kernel,
        out_type=jax.ShapeDtypeStruct((B, S, H), jnp.float32),
        mesh=mesh,
        compiler_params=pltpu.CompilerParams(needs_layout_passes=False),
        scratch_types=[
            pltpu.VMEM((BPW, 2, SHALF), jnp.int32),
            pltpu.VMEM((S, H), jnp.float32),
            pltpu.VMEM((S, H), jnp.float32),
            pltpu.VMEM((S, H), jnp.float32),
            pltpu.VMEM((H,), jnp.float32),
            pltpu.VMEM((H,), jnp.float32),
            pltpu.SemaphoreType.DMA,
            pltpu.SemaphoreType.DMA,
        ],
    )
    def k(ids_hbm, word_hbm, pos_hbm, w_hbm, b_hbm, out_hbm,
          ids_v, buf0, buf1, pos_v, w_v, b_v, gsem, wsem):
        wid = lax.axis_index("s") * NC + lax.axis_index("c")
        base = wid * BPW

        pltpu.sync_copy(ids_hbm.at[pl.ds(base, BPW)], ids_v)
        pltpu.sync_copy(pos_hbm.at[pl.ds(0, S)], pos_v)
        pltpu.sync_copy(w_hbm, w_v)
        pltpu.sync_copy(b_hbm, b_v)

        w_regs = [w_v[pl.ds(j * L, L)] for j in range(HC)]
        b_regs = [b_v[pl.ds(j * L, L)] for j in range(HC)]

        def start_gather(t, buf):
            pltpu.async_copy(
                word_hbm.at[ids_v.at[t, 0]], buf.at[pl.ds(0, SHALF)], gsem)
            pltpu.async_copy(
                word_hbm.at[ids_v.at[t, 1]], buf.at[pl.ds(SHALF, SHALF)], gsem)

        def wait_gather(t, buf):
            pltpu.make_async_copy(
                word_hbm.at[ids_v.at[t, 0]], buf.at[pl.ds(0, SHALF)],
                gsem).wait()
            pltpu.make_async_copy(
                word_hbm.at[ids_v.at[t, 1]], buf.at[pl.ds(SHALF, SHALF)],
                gsem).wait()

        def compute(t, buf):
            @plsc.parallel_loop(0, S, unroll=4)
            def _row(i):
                x = [buf[i, pl.ds(j * L, L)] + pos_v[i, pl.ds(j * L, L)]
                     for j in range(HC)]
                s = (x[0] + x[1]) + (x[2] + x[3])
                s = s + (x[4] + x[5]) + (x[6] + x[7])
                q = x[0] * x[0] + x[1] * x[1]
                q = q + x[2] * x[2] + x[3] * x[3]
                q = q + x[4] * x[4] + x[5] * x[5]
                q = q + x[6] * x[6] + x[7] * x[7]
                tot = jnp.sum(s)
                tot2 = jnp.sum(q)
                mean = tot * jnp.float32(1.0 / H)
                var = tot2 * jnp.float32(1.0 / H) - mean * mean
                inv = _rsqrt_newton(var + jnp.float32(1e-6))
                for j in range(HC):
                    y = (x[j] - mean) * inv * w_regs[j] + b_regs[j]
                    buf[i, pl.ds(j * L, L)] = y

            pltpu.async_copy(buf, out_hbm.at[base + t], wsem)

        def wait_writeback(t, buf):
            pltpu.make_async_copy(buf, out_hbm.at[base + t], wsem).wait()

        start_gather(0, buf0)
        start_gather(1, buf1)
        wait_gather(0, buf0)
        compute(0, buf0)

        def pair(kk, c):
            t1 = 2 * kk + 1
            wait_writeback(t1 - 1, buf0)
            start_gather(t1 + 1, buf0)
            wait_gather(t1, buf1)
            compute(t1, buf1)
            t2 = t1 + 1
            wait_writeback(t2 - 1, buf1)
            start_gather(t2 + 1, buf1)
            wait_gather(t2, buf0)
            compute(t2, buf0)
            return c

        lax.fori_loop(0, (BPW - 2) // 2, pair, 0, unroll=False)

        t_last = BPW - 1
        wait_gather(t_last, buf1)
        compute(t_last, buf1)
        wait_writeback(t_last - 1, buf0)
        wait_writeback(t_last, buf1)

    return k


_kernel_call = _make_kernel()


def kernel(input_ids, word_emb, pos_emb, ln_weight, ln_bias):
    ids3 = input_ids.astype(jnp.int32).reshape(B, 2, SHALF)
    return _kernel_call(ids3, word_emb, pos_emb, ln_weight, ln_bias)
